# baseline (speedup 1.0000x reference)
import functools

import jax
import jax.numpy as jnp
from jax import lax
from jax.experimental import pallas as pl
from jax.experimental.pallas import tpu as pltpu
from jax.experimental.pallas import tpu_sc as plsc

_B = 16384
_D = 32

_info = plsc.get_sparse_core_info()
_NC = _info.num_cores
_NS = _info.num_subcores
_NW = _NC * _NS
_BPW = _B // _NW
_CH = 128
_NCHUNK = _BPW // _CH

_AGE_OFF = 2
_OCC_OFF = 9


def _emb_body(g_hbm, a_hbm, o_hbm, z_hbm, ws_hbm, wz, out,
              ws_v, igv, iav, iov, izv, rz, rows_v,
              gsem0, gsem1, osem0, osem1):
    wid = lax.axis_index("s") * _NC + lax.axis_index("c")
    base = wid * _BPW
    pltpu.sync_copy(ws_hbm, ws_v)
    gsems = (gsem0, gsem1)
    osems = (osem0, osem1)

    def stage_chunk(k):
        b = k % 2
        cbase = base + k * _CH
        pltpu.sync_copy(z_hbm.at[pl.ds(cbase, _CH)], izv.at[b])
        pltpu.sync_copy(g_hbm.at[pl.ds(cbase, _CH)], igv.at[b])
        pltpu.sync_copy(a_hbm.at[pl.ds(cbase, _CH)], iav.at[b])
        pltpu.sync_copy(o_hbm.at[pl.ds(cbase, _CH)], iov.at[b])

        return pltpu.async_copy(wz.at[izv.at[b]], rz.at[b], gsems[b])

    gathers = {0: stage_chunk(0)}
    writes = {}
    for k in range(_NCHUNK):
        b = k % 2
        if k + 1 < _NCHUNK:
            gathers[k + 1] = stage_chunk(k + 1)
        gathers.pop(k).wait()
        if k - 2 in writes:
            writes.pop(k - 2).wait()

        def asm_body(t, _):
            vg = igv[b, pl.ds(t * 16, 16)]
            va = iav[b, pl.ds(t * 16, 16)] + _AGE_OFF
            vo = iov[b, pl.ds(t * 16, 16)] + _OCC_OFF
            for j in range(16):
                i = t * 16 + j
                for c, s in ((0, vg[j]), (1, va[j]), (2, vo[j])):
                    r = s >> 2
                    col = (s & 3) * _D
                    rows_v[b, i, pl.ds(c * _D, 16)] = ws_v[r, pl.ds(col, 16)]
                    rows_v[b, i, pl.ds(c * _D + 16, 16)] = (
                        ws_v[r, pl.ds(col + 16, 16)]
                    )
                rows_v[b, i, pl.ds(3 * _D, 16)] = rz[b, i, pl.ds(0, 16)]
                rows_v[b, i, pl.ds(3 * _D + 16, 16)] = (
                    rz[b, i, pl.ds(16, 16)]
                )
            return ()

        lax.fori_loop(0, _CH // 16, asm_body, ())
        writes[k] = pltpu.async_copy(
            rows_v.at[b], out.at[pl.ds(base + k * _CH, _CH)], osems[b]
        )
    for k in list(writes):
        writes.pop(k).wait()


@jax.jit
def _emb(g, a, o, z, ws, wz):
    mesh = plsc.VectorSubcoreMesh(core_axis_name="c", subcore_axis_name="s")
    f = pl.kernel(
        _emb_body,
        mesh=mesh,
        out_type=jax.ShapeDtypeStruct((_B, 4 * _D), jnp.float32),
        scratch_types=[
            pltpu.VMEM((8, 128), jnp.float32),
            pltpu.VMEM((2, _CH), jnp.int32),
            pltpu.VMEM((2, _CH), jnp.int32),
            pltpu.VMEM((2, _CH), jnp.int32),
            pltpu.VMEM((2, _CH), jnp.int32),
            pltpu.VMEM((2, _CH, 128), jnp.float32),
            pltpu.VMEM((2, _CH, 128), jnp.float32),
            pltpu.SemaphoreType.DMA,
            pltpu.SemaphoreType.DMA,
            pltpu.SemaphoreType.DMA,
            pltpu.SemaphoreType.DMA,
        ],
    )
    return f(g, a, o, z, ws, wz)


def kernel(user_fea, W_gender, W_age, W_occupation, W_area):
    ufi = user_fea.astype(jnp.int32)
    ws = (
        jnp.zeros((32, _D), jnp.float32)
        .at[0:2].set(W_gender)
        .at[_AGE_OFF:_AGE_OFF + 7].set(W_age)
        .at[_OCC_OFF:_OCC_OFF + 21].set(W_occupation)
        .reshape(8, 128)
    )
    wz = jnp.zeros((100000, 128), jnp.float32).at[:, 0:_D].set(W_area)
    return _emb(ufi[:, 0], ufi[:, 1], ufi[:, 2], ufi[:, 3], ws, wz)


# speedup vs baseline: 1.3677x; 1.3677x over previous
import functools

import jax
import jax.numpy as jnp
from jax import lax
from jax.experimental import pallas as pl
from jax.experimental.pallas import tpu as pltpu
from jax.experimental.pallas import tpu_sc as plsc

_B = 16384
_D = 32

_info = plsc.get_sparse_core_info()
_NC = _info.num_cores
_NS = _info.num_subcores
_NW = _NC * _NS
_BPW = _B // _NW
_CH = 128
_NCHUNK = _BPW // _CH

_AGE_OFF = 2
_OCC_OFF = 9


def _emb_body(g_hbm, a_hbm, o_hbm, z_hbm, ws_hbm, wz, out,
              ws_v, igv, iav, iov, izv, rz, rows_v,
              gsem0, gsem1, osem0, osem1):
    wid = lax.axis_index("s") * _NC + lax.axis_index("c")
    base = wid * _BPW
    pltpu.sync_copy(ws_hbm, ws_v)
    gsems = (gsem0, gsem1)
    osems = (osem0, osem1)

    def stage_chunk(k):
        b = k % 2
        cbase = base + k * _CH
        pltpu.sync_copy(z_hbm.at[pl.ds(cbase, _CH)], izv.at[b])
        pltpu.sync_copy(g_hbm.at[pl.ds(cbase, _CH)], igv.at[b])
        pltpu.sync_copy(a_hbm.at[pl.ds(cbase, _CH)], iav.at[b])
        pltpu.sync_copy(o_hbm.at[pl.ds(cbase, _CH)], iov.at[b])

        return pltpu.async_copy(wz.at[izv.at[b]], rz.at[b], gsems[b])

    gathers = {0: stage_chunk(0)}
    writes = {}
    for k in range(_NCHUNK):
        b = k % 2
        if k + 1 < _NCHUNK:
            gathers[k + 1] = stage_chunk(k + 1)
        gathers.pop(k).wait()
        if k - 2 in writes:
            writes.pop(k - 2).wait()

        def asm_body(t, _):
            vg = igv[b, pl.ds(t * 16, 16)]
            va = iav[b, pl.ds(t * 16, 16)] + _AGE_OFF
            vo = iov[b, pl.ds(t * 16, 16)] + _OCC_OFF
            for j in range(16):
                i = t * 16 + j
                for c, s in ((0, vg[j]), (1, va[j]), (2, vo[j])):
                    off = s * _D
                    rows_v[b, i, pl.ds(c * _D, 16)] = ws_v[pl.ds(off, 16)]
                    rows_v[b, i, pl.ds(c * _D + 16, 16)] = (
                        ws_v[pl.ds(off + 16, 16)]
                    )
                rows_v[b, i, pl.ds(3 * _D, 16)] = rz[b, i, pl.ds(0, 16)]
                rows_v[b, i, pl.ds(3 * _D + 16, 16)] = (
                    rz[b, i, pl.ds(16, 16)]
                )
            return ()

        lax.fori_loop(0, _CH // 16, asm_body, ())
        writes[k] = pltpu.async_copy(
            rows_v.at[b], out.at[pl.ds(base + k * _CH, _CH)], osems[b]
        )
    for k in list(writes):
        writes.pop(k).wait()


@jax.jit
def _emb(g, a, o, z, ws, wz):
    mesh = plsc.VectorSubcoreMesh(core_axis_name="c", subcore_axis_name="s")
    f = pl.kernel(
        _emb_body,
        mesh=mesh,
        out_type=jax.ShapeDtypeStruct((_B, 4 * _D), jnp.float32),
        scratch_types=[
            pltpu.VMEM((1024,), jnp.float32),
            pltpu.VMEM((2, _CH), jnp.int32),
            pltpu.VMEM((2, _CH), jnp.int32),
            pltpu.VMEM((2, _CH), jnp.int32),
            pltpu.VMEM((2, _CH), jnp.int32),
            pltpu.VMEM((2, _CH, 128), jnp.float32),
            pltpu.VMEM((2, _CH, 128), jnp.float32),
            pltpu.SemaphoreType.DMA,
            pltpu.SemaphoreType.DMA,
            pltpu.SemaphoreType.DMA,
            pltpu.SemaphoreType.DMA,
        ],
    )
    return f(g, a, o, z, ws, wz)


def kernel(user_fea, W_gender, W_age, W_occupation, W_area):
    ufi = user_fea.astype(jnp.int32)
    ws = (
        jnp.zeros((32, _D), jnp.float32)
        .at[0:2].set(W_gender)
        .at[_AGE_OFF:_AGE_OFF + 7].set(W_age)
        .at[_OCC_OFF:_OCC_OFF + 21].set(W_occupation)
        .reshape(-1)
    )
    wz = jnp.pad(W_area, ((0, 0), (0, 96)))
    return _emb(ufi[:, 0], ufi[:, 1], ufi[:, 2], ufi[:, 3], ws, wz)


# speedup vs baseline: 1.4229x; 1.0404x over previous
import functools

import jax
import jax.numpy as jnp
from jax import lax
from jax.experimental import pallas as pl
from jax.experimental.pallas import tpu as pltpu
from jax.experimental.pallas import tpu_sc as plsc

_B = 16384
_D = 32

_info = plsc.get_sparse_core_info()
_NC = _info.num_cores
_NS = _info.num_subcores
_NW = _NC * _NS
_BPW = _B // _NW
_CH = 256
_NCHUNK = _BPW // _CH

_AGE_OFF = 2
_OCC_OFF = 9


def _emb_body(g_hbm, a_hbm, o_hbm, z_hbm, ws_hbm, wz, out,
              ws_v, igv, iav, iov, izv0, izv1, rz, rows_v,
              gsem0, gsem1):
    wid = lax.axis_index("s") * _NC + lax.axis_index("c")
    base = wid * _BPW
    pltpu.sync_copy(ws_hbm, ws_v)
    gsems = (gsem0, gsem1)
    izvs = (izv0, izv1)

    def stage_chunk(k):
        b = k % 2
        cbase = base + k * _CH
        pltpu.sync_copy(z_hbm.at[pl.ds(cbase, _CH)], izvs[b])
        pltpu.sync_copy(g_hbm.at[pl.ds(cbase, _CH)], igv.at[b])
        pltpu.sync_copy(a_hbm.at[pl.ds(cbase, _CH)], iav.at[b])
        pltpu.sync_copy(o_hbm.at[pl.ds(cbase, _CH)], iov.at[b])

        return pltpu.async_copy(wz.at[izvs[b]], rz.at[b], gsems[b])

    gathers = {0: stage_chunk(0)}
    for k in range(_NCHUNK):
        b = k % 2
        if k + 1 < _NCHUNK:
            gathers[k + 1] = stage_chunk(k + 1)
        gathers.pop(k).wait()

        def asm_body(t, _):
            vg = igv[b, pl.ds(t * 16, 16)]
            va = iav[b, pl.ds(t * 16, 16)] + _AGE_OFF
            vo = iov[b, pl.ds(t * 16, 16)] + _OCC_OFF
            for j in range(16):
                i = t * 16 + j
                for c, s in ((0, vg[j]), (1, va[j]), (2, vo[j])):
                    off = s * _D
                    rows_v[i, pl.ds(c * _D, 16)] = ws_v[pl.ds(off, 16)]
                    rows_v[i, pl.ds(c * _D + 16, 16)] = (
                        ws_v[pl.ds(off + 16, 16)]
                    )
                rows_v[i, pl.ds(3 * _D, 16)] = rz[b, i, pl.ds(0, 16)]
                rows_v[i, pl.ds(3 * _D + 16, 16)] = (
                    rz[b, i, pl.ds(16, 16)]
                )
            return ()

        lax.fori_loop(0, _CH // 16, asm_body, (), unroll=2)
        pltpu.sync_copy(rows_v, out.at[pl.ds(base + k * _CH, _CH)])


@jax.jit
def _emb(g, a, o, z, ws, wz):
    mesh = plsc.VectorSubcoreMesh(core_axis_name="c", subcore_axis_name="s")
    f = pl.kernel(
        _emb_body,
        mesh=mesh,
        out_type=jax.ShapeDtypeStruct((_B, 4 * _D), jnp.float32),
        scratch_types=[
            pltpu.VMEM((1024,), jnp.float32),
            pltpu.VMEM((2, _CH), jnp.int32),
            pltpu.VMEM((2, _CH), jnp.int32),
            pltpu.VMEM((2, _CH), jnp.int32),
            pltpu.VMEM((_CH,), jnp.int32),
            pltpu.VMEM((_CH,), jnp.int32),
            pltpu.VMEM((2, _CH, 128), jnp.float32),
            pltpu.VMEM((_CH, 128), jnp.float32),
            pltpu.SemaphoreType.DMA,
            pltpu.SemaphoreType.DMA,
        ],
    )
    return f(g, a, o, z, ws, wz)


def kernel(user_fea, W_gender, W_age, W_occupation, W_area):
    ufi = user_fea.astype(jnp.int32)
    ws = (
        jnp.zeros((32, _D), jnp.float32)
        .at[0:2].set(W_gender)
        .at[_AGE_OFF:_AGE_OFF + 7].set(W_age)
        .at[_OCC_OFF:_OCC_OFF + 21].set(W_occupation)
        .reshape(-1)
    )
    wz = jnp.pad(W_area, ((0, 0), (0, 96)))
    return _emb(ufi[:, 0], ufi[:, 1], ufi[:, 2], ufi[:, 3], ws, wz)


# speedup vs baseline: 1.4633x; 1.0283x over previous
import functools

import jax
import jax.numpy as jnp
from jax import lax
from jax.experimental import pallas as pl
from jax.experimental.pallas import tpu as pltpu
from jax.experimental.pallas import tpu_sc as plsc

_B = 16384
_D = 32

_info = plsc.get_sparse_core_info()
_NC = _info.num_cores
_NS = _info.num_subcores
_NW = _NC * _NS
_BPW = _B // _NW
_CH = 256
_NCHUNK = _BPW // _CH

_AGE_OFF = 2
_OCC_OFF = 9


def _emb_body(gao_hbm, z_hbm, ws_hbm, wz, out,
              ws_v, igao, izv0, izv1, rz,
              gsem0, gsem1, osem0, osem1):
    wid = lax.axis_index("s") * _NC + lax.axis_index("c")
    base = wid * _BPW
    pltpu.sync_copy(ws_hbm, ws_v)
    gsems = (gsem0, gsem1)
    osems = (osem0, osem1)
    izvs = (izv0, izv1)

    def stage_chunk(k):
        b = k % 2
        cbase = base + k * _CH
        pltpu.sync_copy(z_hbm.at[pl.ds(cbase, _CH)], izvs[b])
        pltpu.sync_copy(gao_hbm.at[:, pl.ds(cbase, _CH)], igao.at[b])

        return pltpu.async_copy(wz.at[izvs[b]], rz.at[b], gsems[b])

    gathers = {0: stage_chunk(0)}
    writes = {}
    for k in range(_NCHUNK):
        b = k % 2
        if k + 1 < _NCHUNK:
            gathers[k + 1] = stage_chunk(k + 1)
        gathers.pop(k).wait()

        def asm_body(t, _):
            vg = igao[b, 0, pl.ds(t * 16, 16)]
            va = igao[b, 1, pl.ds(t * 16, 16)] + _AGE_OFF
            vo = igao[b, 2, pl.ds(t * 16, 16)] + _OCC_OFF
            for j in range(16):
                i = t * 16 + j
                z0 = rz[b, i, pl.ds(0, 16)]
                z1 = rz[b, i, pl.ds(16, 16)]
                for c, s in ((0, vg[j]), (1, va[j]), (2, vo[j])):
                    off = s * _D
                    rz[b, i, pl.ds(c * _D, 16)] = ws_v[pl.ds(off, 16)]
                    rz[b, i, pl.ds(c * _D + 16, 16)] = (
                        ws_v[pl.ds(off + 16, 16)]
                    )
                rz[b, i, pl.ds(3 * _D, 16)] = z0
                rz[b, i, pl.ds(3 * _D + 16, 16)] = z1
            return ()

        lax.fori_loop(0, _CH // 16, asm_body, (), unroll=2)
        writes[k] = pltpu.async_copy(
            rz.at[b], out.at[pl.ds(base + k * _CH, _CH)], osems[b]
        )
    for k in list(writes):
        writes.pop(k).wait()


@jax.jit
def _emb(gao, z, ws, wz):
    mesh = plsc.VectorSubcoreMesh(core_axis_name="c", subcore_axis_name="s")
    f = pl.kernel(
        _emb_body,
        mesh=mesh,
        out_type=jax.ShapeDtypeStruct((_B, 4 * _D), jnp.float32),
        scratch_types=[
            pltpu.VMEM((1024,), jnp.float32),
            pltpu.VMEM((2, 3, _CH), jnp.int32),
            pltpu.VMEM((_CH,), jnp.int32),
            pltpu.VMEM((_CH,), jnp.int32),
            pltpu.VMEM((2, _CH, 128), jnp.float32),
            pltpu.SemaphoreType.DMA,
            pltpu.SemaphoreType.DMA,
            pltpu.SemaphoreType.DMA,
            pltpu.SemaphoreType.DMA,
        ],
    )
    return f(gao, z, ws, wz)


def kernel(user_fea, W_gender, W_age, W_occupation, W_area):
    ufi = user_fea.astype(jnp.int32)
    ws = (
        jnp.zeros((32, _D), jnp.float32)
        .at[0:2].set(W_gender)
        .at[_AGE_OFF:_AGE_OFF + 7].set(W_age)
        .at[_OCC_OFF:_OCC_OFF + 21].set(W_occupation)
        .reshape(-1)
    )
    wz = jnp.pad(W_area, ((0, 0), (0, 96)))
    gao = jnp.stack([ufi[:, 0], ufi[:, 1], ufi[:, 2]])
    return _emb(gao, ufi[:, 3], ws, wz)
